# baseline (speedup 1.0000x reference)
import jax
import jax.numpy as jnp
from jax.experimental import pallas as pl


def kernel(pts, tex, edges, mem):
    edges2 = jnp.concatenate([edges, jnp.stack([edges[:, 1], edges[:, 0]], axis=1)], axis=0)
    l_tex = jnp.clip(jnp.round(tex), 0.0, 1.0).astype(jnp.int32)
    tex_lf = jnp.take(l_tex, edges2[:, 0], axis=0)
    tex_rt = jnp.take(l_tex, edges2[:, 1], axis=0)
    locs = pts[:, :2]
    locs_lf = jnp.take(locs, edges2[:, 0], axis=0)
    locs_rt = jnp.take(locs, edges2[:, 1], axis=0)
    rel_vec = jnp.concatenate([locs_lf, locs_rt], axis=1) + 32.0
    rel_vec = jnp.clip(jnp.round(rel_vec), 0.0, 64.0).astype(jnp.int32)
    new_mem = mem.at[rel_vec[:, 0], rel_vec[:, 1], tex_lf,
                     rel_vec[:, 2], rel_vec[:, 3], tex_rt].add(
        jnp.ones((rel_vec.shape[0],), dtype=jnp.float32))
    return new_mem


# speedup vs baseline: 1.7222x; 1.7222x over previous
import functools

import jax
import jax.numpy as jnp
from jax import lax
from jax.experimental import pallas as pl
from jax.experimental.pallas import tpu as pltpu
from jax.experimental.pallas import tpu_sc as plsc

NN = 100000
NE = 1600000
CODES = 8450
NB = 65 * 65 * 2 * 65 * 65 * 2

NC, NS = 2, 16
NW = NC * NS

EPW = NE // NW
ECH = 2000
NECH = EPW // ECH
ACH = 2000
NACH = NN // ACH
FPW = 102400
NF = NW * FPW

BCH = 4096
NBCH = 29
TSL = BCH * NBCH
W = TSL * NS
NPASS = -(-NB // (2 * W))
NBP = NPASS * 2 * W
HSZ = W + 64
SCH = 2048
FPT = NF // NS
NSCH = FPT // SCH

_mesh = plsc.VectorSubcoreMesh(core_axis_name="c", subcore_axis_name="s")


def _quant(y):
    r = (y + 0.5).astype(jnp.int32)
    rf = r.astype(jnp.float32)
    tie = (rf - y) == 0.5
    odd = (r & 1) == 1
    r = r - jnp.where(tie & odd, 1, 0)
    return jnp.clip(r, 0, 64)


def _k1_body(xs, ys, tex, ea, eb, flats, lv, xsv, ysv, txv, eav, ebv, f1v, f2v):
    wid = lax.axis_index("s") * NC + lax.axis_index("c")

    def a_chunk(ci, _):
        base = ci * ACH
        pltpu.sync_copy(xs.at[pl.ds(base, ACH)], xsv)
        pltpu.sync_copy(ys.at[pl.ds(base, ACH)], ysv)
        pltpu.sync_copy(tex.at[pl.ds(base, ACH)], txv)

        def vec(i, _):
            o = i * 16
            qx = _quant(xsv[pl.ds(o, 16)] + 32.0)
            qy = _quant(ysv[pl.ds(o, 16)] + 32.0)
            t = txv[pl.ds(o, 16)]
            tq = jnp.where(t >= 0.5, 1, 0).astype(jnp.int32)
            tq = jnp.where(t == 0.5, 0, tq)
            lv[pl.ds(base + o, 16)] = (qx * 65 + qy) * 2 + tq
            return 0

        lax.fori_loop(0, ACH // 16, vec, 0)
        return 0

    lax.fori_loop(0, NACH, a_chunk, 0)

    ebase = wid * EPW
    fbase = wid * FPW

    def b_chunk(ci, _):
        off = ci * ECH
        pltpu.sync_copy(ea.at[pl.ds(ebase + off, ECH)], eav)
        pltpu.sync_copy(eb.at[pl.ds(ebase + off, ECH)], ebv)

        def vec(i, _):
            o = i * 16
            a = eav[pl.ds(o, 16)]
            b = ebv[pl.ds(o, 16)]
            la = plsc.load_gather(lv, [a])
            lb = plsc.load_gather(lv, [b])
            f1v[pl.ds(o, 16)] = la * CODES + lb
            f2v[pl.ds(o, 16)] = lb * CODES + la
            return 0

        lax.fori_loop(0, ECH // 16, vec, 0)
        pltpu.sync_copy(f1v, flats.at[pl.ds(fbase + off, ECH)])
        pltpu.sync_copy(f2v, flats.at[pl.ds(fbase + EPW + off, ECH)])
        return 0

    lax.fori_loop(0, NECH, b_chunk, 0)

    neg1 = jnp.full((16,), -1, jnp.int32)

    def fill(i, _):
        f1v[pl.ds(i * 16, 16)] = neg1
        return 0

    lax.fori_loop(0, ECH // 16, fill, 0)
    pltpu.sync_copy(f1v, flats.at[pl.ds(fbase + 2 * EPW, ECH)])
    pltpu.sync_copy(f1v.at[pl.ds(0, 400)],
                    flats.at[pl.ds(fbase + 2 * EPW + ECH, 400)])


def _k2_body(flats, memf, outf, fv, onesv, bounce, hist, sem):
    core = lax.axis_index("c")
    sid = lax.axis_index("s")
    lane = lax.broadcasted_iota(jnp.int32, (16,), 0)

    def ones_init(i, _):
        onesv[pl.ds(i * 16, 16)] = jnp.full((16,), 1.0, jnp.float32)
        return 0

    lax.fori_loop(0, SCH // 16, ones_init, 0)

    def do_pass(p, _):
        wb = (p * 2 + core) * W

        def load_j(j, _):
            pltpu.sync_copy(memf.at[pl.ds(wb + sid * TSL + j * BCH, BCH)],
                            bounce)
            pltpu.sync_copy(bounce, hist.at[pl.ds(sid * TSL + j * BCH, BCH)])
            return 0

        lax.fori_loop(0, NBCH, load_j, 0)
        plsc.subcore_barrier()

        def chunk(ci, _):
            src = sid * FPT + ci * SCH
            pltpu.sync_copy(flats.at[pl.ds(src, SCH)], fv)

            def vec(i, _):
                v = fv[pl.ds(i * 16, 16)]
                rel = v - wb
                ok = (rel >= 0) & (rel < W)
                fv[pl.ds(i * 16, 16)] = jnp.where(ok, rel, W + lane)
                return 0

            lax.fori_loop(0, SCH // 16, vec, 0)
            pltpu.async_copy(onesv, hist.at[fv], sem, add=True).wait()
            return 0

        lax.fori_loop(0, NSCH, chunk, 0)
        plsc.subcore_barrier()

        def store_j(j, _):
            pltpu.sync_copy(hist.at[pl.ds(sid * TSL + j * BCH, BCH)], bounce)
            pltpu.sync_copy(bounce, outf.at[pl.ds(wb + sid * TSL + j * BCH, BCH)])
            return 0

        lax.fori_loop(0, NBCH, store_j, 0)
        return 0

    lax.fori_loop(0, NPASS, do_pass, 0)


_params = pltpu.CompilerParams(needs_layout_passes=False)

_k1 = pl.kernel(
    _k1_body,
    out_type=jax.ShapeDtypeStruct((NF,), jnp.int32),
    mesh=_mesh,
    compiler_params=_params,
    scratch_types=[
        pltpu.VMEM((NN,), jnp.int32),
        pltpu.VMEM((ACH,), jnp.float32),
        pltpu.VMEM((ACH,), jnp.float32),
        pltpu.VMEM((ACH,), jnp.float32),
        pltpu.VMEM((ECH,), jnp.int32),
        pltpu.VMEM((ECH,), jnp.int32),
        pltpu.VMEM((ECH,), jnp.int32),
        pltpu.VMEM((ECH,), jnp.int32),
    ],
)

_k2 = pl.kernel(
    _k2_body,
    out_type=jax.ShapeDtypeStruct((NBP,), jnp.float32),
    mesh=_mesh,
    compiler_params=_params,
    scratch_types=[
        pltpu.VMEM((SCH,), jnp.int32),
        pltpu.VMEM((SCH,), jnp.float32),
        pltpu.VMEM((BCH,), jnp.float32),
        pltpu.VMEM_SHARED((HSZ,), jnp.float32),
        pltpu.SemaphoreType.DMA,
    ],
)


@jax.jit
def kernel(pts, tex, edges, mem):
    xs = pts[:, 0]
    ys = pts[:, 1]
    ea = edges[:, 0]
    eb = edges[:, 1]
    memf = jnp.concatenate(
        [mem.reshape(NB), jnp.zeros(NBP - NB, jnp.float32)])
    flats = _k1(xs, ys, tex, ea, eb)
    outp = _k2(flats, memf)
    return outp[:NB].reshape(mem.shape)


# speedup vs baseline: 7.6984x; 4.4702x over previous
import jax
import jax.numpy as jnp
from jax import lax
from jax.experimental import pallas as pl
from jax.experimental.pallas import tpu as pltpu
from jax.experimental.pallas import tpu_sc as plsc

NN = 100000
NE = 1600000
CODES = 8450
NB = CODES * CODES
MEM_SHAPE = (65, 65, 2, 65, 65, 2)
TSHAPE = (65, 65, 2, 65, 2, 65)
PERM = (0, 1, 2, 3, 5, 4)

NC, NS = 2, 16

EPW = NE // (NC * NS)
ECH = 2000
NECH = EPW // ECH
ACH = 2000
NACH = NN // ACH
FPW = 102400
NF = NC * NS * FPW

BCH = 8192
NBCH = 14
TSL = BCH * NBCH
W = TSL * NS
NPASS = -(-NB // (2 * W))
HSZ = W + 64
SCH = 2048
FPT = NF // NS
NSCH = FPT // SCH
TAIL = NB - (NPASS - 1) * 2 * W - 13 * TSL - 8 * BCH - 114688

_mesh = plsc.VectorSubcoreMesh(core_axis_name="c", subcore_axis_name="s")
_params = pltpu.CompilerParams(needs_layout_passes=False)


def _quant(y):
    r = (y + 0.5).astype(jnp.int32)
    rf = r.astype(jnp.float32)
    tie = (rf - y) == 0.5
    odd = (r & 1) == 1
    r = r - jnp.where(tie & odd, 1, 0)
    return jnp.clip(r, 0, 64)


def _k1_body(xs, ys, tex, ea, eb, flats, pv, xsv, ysv, txv, eav, ebv, f1v, f2v):
    wid = lax.axis_index("s") * NC + lax.axis_index("c")

    def a_chunk(ci, _):
        base = ci * ACH
        pltpu.sync_copy(xs.at[pl.ds(base, ACH)], xsv)
        pltpu.sync_copy(ys.at[pl.ds(base, ACH)], ysv)
        pltpu.sync_copy(tex.at[pl.ds(base, ACH)], txv)

        def vec(i, _):
            o = i * 16
            qx = _quant(xsv[pl.ds(o, 16)] + 32.0)
            qy = _quant(ysv[pl.ds(o, 16)] + 32.0)
            t = txv[pl.ds(o, 16)]
            tq = jnp.where(t >= 0.5, 1, 0).astype(jnp.int32)
            tq = jnp.where(t == 0.5, 0, tq)
            lcode = (qx * 65 + qy) * 2 + tq
            rcode = (qx * 2 + tq) * 65 + qy
            pv[pl.ds(base + o, 16)] = lcode * 16384 + rcode
            return 0

        lax.fori_loop(0, ACH // 16, vec, 0)
        return 0

    lax.fori_loop(0, NACH, a_chunk, 0)

    ebase = wid * EPW
    fbase = wid * FPW

    def b_chunk(ci, _):
        off = ci * ECH
        pltpu.sync_copy(ea.at[pl.ds(ebase + off, ECH)], eav)
        pltpu.sync_copy(eb.at[pl.ds(ebase + off, ECH)], ebv)

        def vec(i, _):
            o = i * 16
            pa = plsc.load_gather(pv, [eav[pl.ds(o, 16)]])
            pb = plsc.load_gather(pv, [ebv[pl.ds(o, 16)]])
            f1v[pl.ds(o, 16)] = (pa >> 14) * CODES + (pb & 16383)
            f2v[pl.ds(o, 16)] = (pb >> 14) * CODES + (pa & 16383)
            return 0

        lax.fori_loop(0, ECH // 16, vec, 0)
        pltpu.sync_copy(f1v, flats.at[pl.ds(fbase + off, ECH)])
        pltpu.sync_copy(f2v, flats.at[pl.ds(fbase + EPW + off, ECH)])
        return 0

    lax.fori_loop(0, NECH, b_chunk, 0)

    neg1 = jnp.full((16,), -1, jnp.int32)

    def fill(i, _):
        f1v[pl.ds(i * 16, 16)] = neg1
        return 0

    lax.fori_loop(0, ECH // 16, fill, 0)
    pltpu.sync_copy(f1v, flats.at[pl.ds(fbase + 2 * EPW, ECH)])
    pltpu.sync_copy(f1v.at[pl.ds(0, 400)],
                    flats.at[pl.ds(fbase + 2 * EPW + ECH, 400)])


def _k2_body(flats, memf, outf, fv, onesv, bounce, hist, sem):
    core = lax.axis_index("c")
    sid = lax.axis_index("s")
    lane = lax.broadcasted_iota(jnp.int32, (16,), 0)

    def ones_init(i, _):
        onesv[pl.ds(i * 16, 16)] = jnp.full((16,), 1.0, jnp.float32)
        return 0

    lax.fori_loop(0, SCH // 16, ones_init, 0)

    def do_pass(p, _):
        wb = (p * 2 + core) * W

        def move(j, to_hbm):
            start = wb + sid * TSL + j * BCH
            full = start + BCH <= NB
            part = jnp.logical_and(start < NB, jnp.logical_not(full))

            @pl.when(full)
            def _():
                if to_hbm:
                    pltpu.sync_copy(hist.at[pl.ds(sid * TSL + j * BCH, BCH)],
                                    bounce)
                    pltpu.sync_copy(bounce, outf.at[pl.ds(start, BCH)])
                else:
                    pltpu.sync_copy(memf.at[pl.ds(start, BCH)], bounce)
                    pltpu.sync_copy(bounce,
                                    hist.at[pl.ds(sid * TSL + j * BCH, BCH)])

            @pl.when(part)
            def _():
                if to_hbm:
                    pltpu.sync_copy(hist.at[pl.ds(sid * TSL + j * BCH, TAIL)],
                                    bounce.at[pl.ds(0, TAIL)])
                    pltpu.sync_copy(bounce.at[pl.ds(0, TAIL)],
                                    outf.at[pl.ds(start, TAIL)])
                else:
                    pltpu.sync_copy(memf.at[pl.ds(start, TAIL)],
                                    bounce.at[pl.ds(0, TAIL)])
                    pltpu.sync_copy(bounce.at[pl.ds(0, TAIL)],
                                    hist.at[pl.ds(sid * TSL + j * BCH, TAIL)])

        def load_j(j, _):
            move(j, False)
            return 0

        lax.fori_loop(0, NBCH, load_j, 0)
        plsc.subcore_barrier()

        @pl.when(wb < NB)
        def _():
            def chunk(ci, _):
                src = sid * FPT + ci * SCH
                pltpu.sync_copy(flats.at[pl.ds(src, SCH)], fv)

                def vec(i, _):
                    v = fv[pl.ds(i * 16, 16)]
                    rel = v - wb
                    ok = (rel >= 0) & (rel < W)
                    fv[pl.ds(i * 16, 16)] = jnp.where(ok, rel, W + lane)
                    return 0

                lax.fori_loop(0, SCH // 16, vec, 0)
                pltpu.async_copy(onesv, hist.at[fv], sem, add=True).wait()
                return 0

            lax.fori_loop(0, NSCH, chunk, 0)

        plsc.subcore_barrier()

        def store_j(j, _):
            move(j, True)
            return 0

        lax.fori_loop(0, NBCH, store_j, 0)
        return 0

    lax.fori_loop(0, NPASS, do_pass, 0)


_k1 = pl.kernel(
    _k1_body,
    out_type=jax.ShapeDtypeStruct((NF,), jnp.int32),
    mesh=_mesh,
    compiler_params=_params,
    scratch_types=[
        pltpu.VMEM((NN,), jnp.int32),
        pltpu.VMEM((ACH,), jnp.float32),
        pltpu.VMEM((ACH,), jnp.float32),
        pltpu.VMEM((ACH,), jnp.float32),
        pltpu.VMEM((ECH,), jnp.int32),
        pltpu.VMEM((ECH,), jnp.int32),
        pltpu.VMEM((ECH,), jnp.int32),
        pltpu.VMEM((ECH,), jnp.int32),
    ],
)

_k2 = pl.kernel(
    _k2_body,
    out_type=jax.ShapeDtypeStruct((NB,), jnp.float32),
    mesh=_mesh,
    compiler_params=_params,
    scratch_types=[
        pltpu.VMEM((SCH,), jnp.int32),
        pltpu.VMEM((SCH,), jnp.float32),
        pltpu.VMEM((BCH,), jnp.float32),
        pltpu.VMEM_SHARED((HSZ,), jnp.float32),
        pltpu.SemaphoreType.DMA,
    ],
)


@jax.jit
def kernel(pts, tex, edges, mem):
    xs = pts[:, 0]
    ys = pts[:, 1]
    ea = edges[:, 0]
    eb = edges[:, 1]
    memf = mem.transpose(PERM).reshape(-1)
    flats = _k1(xs, ys, tex, ea, eb)
    outp = _k2(flats, memf)
    return outp.reshape(TSHAPE).transpose(PERM)


# speedup vs baseline: 7.7037x; 1.0007x over previous
import jax
import jax.numpy as jnp
from jax import lax
from jax.experimental import pallas as pl
from jax.experimental.pallas import tpu as pltpu
from jax.experimental.pallas import tpu_sc as plsc

NN = 100000
NE = 1600000
CODES = 8450
NB = CODES * CODES
MEM_SHAPE = (65, 65, 2, 65, 65, 2)
TSHAPE = (65, 65, 2, 65, 2, 65)
PERM = (0, 1, 2, 3, 5, 4)

NC, NS = 2, 16

EPW = NE // (NC * NS)
ECH = 2000
NECH = EPW // ECH
ACH = 2000
NACH = NN // ACH
FPW = 102400
NF = NC * NS * FPW

BCH = 7168
NBCH = 16
TSL = BCH * NBCH
W = TSL * NS
NPASS = -(-NB // (2 * W))
HSZ = W + 64
SCH = 2048
FPT = NF // NS
NSCH = FPT // SCH
TAIL = NB - (NPASS - 1) * 2 * W - 14 * TSL - 9 * BCH

_mesh = plsc.VectorSubcoreMesh(core_axis_name="c", subcore_axis_name="s")
_params = pltpu.CompilerParams(needs_layout_passes=False)


def _quant(y):
    r = (y + 0.5).astype(jnp.int32)
    rf = r.astype(jnp.float32)
    tie = (rf - y) == 0.5
    odd = (r & 1) == 1
    r = r - jnp.where(tie & odd, 1, 0)
    return jnp.clip(r, 0, 64)


def _k1_body(xs, ys, tex, ea, eb, flats, pv, xsv, ysv, txv, eav, ebv, f1v, f2v):
    wid = lax.axis_index("s") * NC + lax.axis_index("c")

    def a_chunk(ci, _):
        base = ci * ACH
        pltpu.sync_copy(xs.at[pl.ds(base, ACH)], xsv)
        pltpu.sync_copy(ys.at[pl.ds(base, ACH)], ysv)
        pltpu.sync_copy(tex.at[pl.ds(base, ACH)], txv)

        def vec(i, _):
            o = i * 16
            qx = _quant(xsv[pl.ds(o, 16)] + 32.0)
            qy = _quant(ysv[pl.ds(o, 16)] + 32.0)
            t = txv[pl.ds(o, 16)]
            tq = jnp.where(t >= 0.5, 1, 0).astype(jnp.int32)
            tq = jnp.where(t == 0.5, 0, tq)
            lcode = (qx * 65 + qy) * 2 + tq
            rcode = (qx * 2 + tq) * 65 + qy
            pv[pl.ds(base + o, 16)] = lcode * 16384 + rcode
            return 0

        lax.fori_loop(0, ACH // 16, vec, 0)
        return 0

    lax.fori_loop(0, NACH, a_chunk, 0)

    ebase = wid * EPW
    fbase = wid * FPW

    def b_chunk(ci, _):
        off = ci * ECH
        pltpu.sync_copy(ea.at[pl.ds(ebase + off, ECH)], eav)
        pltpu.sync_copy(eb.at[pl.ds(ebase + off, ECH)], ebv)

        def vec(i, _):
            o = i * 16
            pa = plsc.load_gather(pv, [eav[pl.ds(o, 16)]])
            pb = plsc.load_gather(pv, [ebv[pl.ds(o, 16)]])
            f1v[pl.ds(o, 16)] = (pa >> 14) * CODES + (pb & 16383)
            f2v[pl.ds(o, 16)] = (pb >> 14) * CODES + (pa & 16383)
            return 0

        lax.fori_loop(0, ECH // 16, vec, 0)
        pltpu.sync_copy(f1v, flats.at[pl.ds(fbase + off, ECH)])
        pltpu.sync_copy(f2v, flats.at[pl.ds(fbase + EPW + off, ECH)])
        return 0

    lax.fori_loop(0, NECH, b_chunk, 0)

    neg1 = jnp.full((16,), -1, jnp.int32)

    def fill(i, _):
        f1v[pl.ds(i * 16, 16)] = neg1
        return 0

    lax.fori_loop(0, ECH // 16, fill, 0)
    pltpu.sync_copy(f1v, flats.at[pl.ds(fbase + 2 * EPW, ECH)])
    pltpu.sync_copy(f1v.at[pl.ds(0, 400)],
                    flats.at[pl.ds(fbase + 2 * EPW + ECH, 400)])


def _k2_body(flats, memf, outf, fv0, fv1, fv2, onesv, bounce, hist, sem,
             dsem, ssem):
    core = lax.axis_index("c")
    sid = lax.axis_index("s")
    lane = lax.broadcasted_iota(jnp.int32, (16,), 0)

    def ones_init(i, _):
        onesv[pl.ds(i * 16, 16)] = jnp.full((16,), 1.0, jnp.float32)
        return 0

    lax.fori_loop(0, SCH // 16, ones_init, 0)

    def do_pass(p, _):
        wb = (p * 2 + core) * W

        def move(j, to_hbm):
            start = wb + sid * TSL + j * BCH
            full = start + BCH <= NB
            part = jnp.logical_and(start < NB, jnp.logical_not(full))

            @pl.when(full)
            def _():
                if to_hbm:
                    pltpu.sync_copy(hist.at[pl.ds(sid * TSL + j * BCH, BCH)],
                                    bounce)
                    pltpu.sync_copy(bounce, outf.at[pl.ds(start, BCH)])
                else:
                    pltpu.sync_copy(memf.at[pl.ds(start, BCH)], bounce)
                    pltpu.sync_copy(bounce,
                                    hist.at[pl.ds(sid * TSL + j * BCH, BCH)])

            @pl.when(part)
            def _():
                if to_hbm:
                    pltpu.sync_copy(hist.at[pl.ds(sid * TSL + j * BCH, TAIL)],
                                    bounce.at[pl.ds(0, TAIL)])
                    pltpu.sync_copy(bounce.at[pl.ds(0, TAIL)],
                                    outf.at[pl.ds(start, TAIL)])
                else:
                    pltpu.sync_copy(memf.at[pl.ds(start, TAIL)],
                                    bounce.at[pl.ds(0, TAIL)])
                    pltpu.sync_copy(bounce.at[pl.ds(0, TAIL)],
                                    hist.at[pl.ds(sid * TSL + j * BCH, TAIL)])

        def load_j(j, _):
            move(j, False)
            return 0

        lax.fori_loop(0, NBCH, load_j, 0)
        plsc.subcore_barrier()

        @pl.when(wb < NB)
        def _():
            base = sid * FPT
            bufs = (fv0, fv1, fv2)

            def scan_buf(buf):
                def vec(i, _):
                    for u8 in range(8):
                        o = (i * 8 + u8) * 16
                        v = buf[pl.ds(o, 16)]
                        rel = v - wb
                        ok = (rel >= 0) & (rel < W)
                        buf[pl.ds(o, 16)] = jnp.where(ok, rel, W + lane)
                    return 0

                lax.fori_loop(0, SCH // 128, vec, 0)

            def dma_start(ci, buf):
                pltpu.async_copy(flats.at[pl.ds(base + ci * SCH, SCH)], buf,
                                 dsem)

            def dma_drain(ci, buf):
                pltpu.make_async_copy(flats.at[pl.ds(base + ci * SCH, SCH)],
                                      buf, dsem).wait()

            def sc_fire(buf):
                pltpu.async_copy(onesv, hist.at[buf], ssem, add=True)

            def sc_drain(buf):
                pltpu.make_async_copy(onesv, hist.at[buf], ssem).wait()

            dma_start(0, fv0)
            dma_start(1, fv1)

            def super_chunk(g, _):
                for u in range(3):
                    buf = bufs[u]
                    pbuf = bufs[(u + 2) % 3]
                    ci = g * 3 + u
                    dma_drain(ci, buf)
                    scan_buf(buf)
                    sc_fire(buf)
                    if u == 0:
                        @pl.when(g > 0)
                        def _():
                            sc_drain(pbuf)

                        dma_start(ci + 2, pbuf)
                    elif u == 1:
                        sc_drain(pbuf)
                        dma_start(ci + 2, pbuf)
                    else:
                        sc_drain(pbuf)

                        @pl.when(g < (NSCH // 3) - 1)
                        def _():
                            dma_start(ci + 2, pbuf)
                return 0

            lax.fori_loop(0, NSCH // 3, super_chunk, 0)
            dma_drain(NSCH - 1, fv0)
            scan_buf(fv0)
            sc_fire(fv0)
            sc_drain(fv2)
            sc_drain(fv0)

        plsc.subcore_barrier()

        def store_j(j, _):
            move(j, True)
            return 0

        lax.fori_loop(0, NBCH, store_j, 0)
        return 0

    lax.fori_loop(0, NPASS, do_pass, 0)


_k1 = pl.kernel(
    _k1_body,
    out_type=jax.ShapeDtypeStruct((NF,), jnp.int32),
    mesh=_mesh,
    compiler_params=_params,
    scratch_types=[
        pltpu.VMEM((NN,), jnp.int32),
        pltpu.VMEM((ACH,), jnp.float32),
        pltpu.VMEM((ACH,), jnp.float32),
        pltpu.VMEM((ACH,), jnp.float32),
        pltpu.VMEM((ECH,), jnp.int32),
        pltpu.VMEM((ECH,), jnp.int32),
        pltpu.VMEM((ECH,), jnp.int32),
        pltpu.VMEM((ECH,), jnp.int32),
    ],
)

_k2 = pl.kernel(
    _k2_body,
    out_type=jax.ShapeDtypeStruct((NB,), jnp.float32),
    mesh=_mesh,
    compiler_params=_params,
    scratch_types=[
        pltpu.VMEM((SCH,), jnp.int32),
        pltpu.VMEM((SCH,), jnp.int32),
        pltpu.VMEM((SCH,), jnp.int32),
        pltpu.VMEM((SCH,), jnp.float32),
        pltpu.VMEM((BCH,), jnp.float32),
        pltpu.VMEM_SHARED((HSZ,), jnp.float32),
        pltpu.SemaphoreType.DMA,
        pltpu.SemaphoreType.DMA,
        pltpu.SemaphoreType.DMA,
    ],
)


@jax.jit
def kernel(pts, tex, edges, mem):
    xs = pts[:, 0]
    ys = pts[:, 1]
    ea = edges[:, 0]
    eb = edges[:, 1]
    memf = mem.transpose(PERM).reshape(-1)
    flats = _k1(xs, ys, tex, ea, eb)
    outp = _k2(flats, memf)
    return outp.reshape(TSHAPE).transpose(PERM)
